# baseline (speedup 1.0000x reference)
import functools

import jax
import jax.numpy as jnp
from jax import lax
from jax.experimental import pallas as pl
from jax.experimental.pallas import tpu as pltpu
from jax.experimental.pallas import tpu_sc as plsc

_P_SA0 = 0.1
_P_SA1 = 0.1
_G_SA0 = 0.003
_G_SA1 = 3e-06

_M = 16384
_N = 4096
_NC = 2
_NS = 16
_NW = _NC * _NS
_ROWS_W = _M // _NW
_CR = 8
_CN = 2048
_CSTEPS_R = _ROWS_W // _CR
_CSTEPS_N = _N // _CN
_STEPS = _CSTEPS_R * _CSTEPS_N
_LANES = 16
_NBUF = 3


def _saf_chunk(xb, pb, b):
    iota = jnp.arange(_LANES, dtype=jnp.int32)
    bv = jnp.full((_LANES,), b, dtype=jnp.int32)
    for r in range(_CR):
        rv = jnp.full((_LANES,), r, dtype=jnp.int32)

        @plsc.parallel_loop(0, _CN, step=_LANES, unroll=8)
        def _(c):
            pv = pb[r, pl.ds(c, _LANES)]
            le = pv <= jnp.float32(_P_SA0)
            gt = pv > jnp.float32(1.0 - _P_SA1)
            val = jnp.where(le, jnp.float32(_G_SA0), jnp.float32(_G_SA1))
            plsc.store_scatter(xb, [bv, rv, iota + c], val, mask=le | gt)


_mesh = plsc.VectorSubcoreMesh(core_axis_name="c", subcore_axis_name="s")


@functools.partial(
    pl.kernel,
    mesh=_mesh,
    out_type=jax.ShapeDtypeStruct((_M, _N), jnp.float32),
    scratch_types=[
        pltpu.VMEM((_NBUF, _CR, _CN), jnp.float32),
        pltpu.VMEM((_NBUF, _CR, _CN), jnp.float32),
        pltpu.SemaphoreType.DMA((_NBUF,)),
        pltpu.SemaphoreType.DMA((_NBUF,)),
        pltpu.SemaphoreType.DMA((_NBUF,)),
    ],
    compiler_params=pltpu.CompilerParams(needs_layout_passes=False),
)
def _saf_sc(x_hbm, p_hbm, o_hbm, xb, pb, lx_sem, lp_sem, st_sem):
    wid = lax.axis_index("s") * _NC + lax.axis_index("c")
    base = wid * _ROWS_W

    def chunk_slice(s):
        row = base + lax.div(s, _CSTEPS_N) * _CR
        col = lax.rem(s, _CSTEPS_N) * _CN
        return (pl.ds(row, _CR), pl.ds(col, _CN))

    def load(s, b):
        sl = chunk_slice(s)
        pltpu.make_async_copy(x_hbm.at[sl[0], sl[1]], xb.at[b], lx_sem.at[b]).start()
        pltpu.make_async_copy(p_hbm.at[sl[0], sl[1]], pb.at[b], lp_sem.at[b]).start()

    def wait_store(s, b):
        sl = chunk_slice(s)
        pltpu.make_async_copy(xb.at[b], o_hbm.at[sl[0], sl[1]], st_sem.at[b]).wait()

    load(0, 0)
    load(1, 1)

    def step(s, _):
        b = lax.rem(s, _NBUF)
        sl = chunk_slice(s)
        pltpu.make_async_copy(x_hbm.at[sl[0], sl[1]], xb.at[b], lx_sem.at[b]).wait()
        pltpu.make_async_copy(p_hbm.at[sl[0], sl[1]], pb.at[b], lp_sem.at[b]).wait()

        _saf_chunk(xb, pb.at[b], b)
        pltpu.make_async_copy(xb.at[b], o_hbm.at[sl[0], sl[1]], st_sem.at[b]).start()

        @pl.when(s + 2 < _STEPS)
        def _():
            b2 = lax.rem(s + 2, _NBUF)

            @pl.when(s >= 1)
            def _():
                wait_store(s - 1, b2)

            load(s + 2, b2)

        return 0

    lax.fori_loop(0, _STEPS, step, 0)

    for s in range(_STEPS - 3, _STEPS):
        wait_store(s, s % _NBUF)


def kernel(input, p_state):
    return _saf_sc(input, p_state)
